# Initial kernel scaffold; baseline (speedup 1.0000x reference)
import jax
import jax.numpy as jnp
from jax.experimental import pallas as pl


def kernel(x, rotation_reg, S_reg, rotation_out, S_out):
    raise NotImplementedError("write your pallas kernel here")


# speedup vs baseline: 240.2948x; 240.2948x over previous
import math

import jax
import jax.numpy as jnp
from jax.experimental import pallas as pl

_DIM = 128
_NOUT = 32
_NREG = _DIM - _NOUT

_L1 = 0.7978845608
_L2A = 0.4527800
_L2B = 1.5104176
_MID2 = (_L2A + _L2B) / 2.0
_K_REG = math.sqrt(math.pi / 2.0) / _NREG
_K_OUT = math.sqrt(math.pi / 2.0) / _NOUT


def _dot(a, b, prec):
    return jax.lax.dot_general(a, b, (((1,), (0,)), ((), ())),
                               preferred_element_type=jnp.float32,
                               precision=prec)


def _dot_t(a, b, prec):
    return jax.lax.dot_general(a, b, (((1,), (1,)), ((), ())),
                               preferred_element_type=jnp.float32,
                               precision=prec)


_EXACT = jax.lax.Precision.HIGHEST
_MM = jax.lax.Precision.DEFAULT


def _tq_kernel(x_ref, rbd_ref, rbdt_ref, sbd_ref, o_ref):
    f32 = jnp.float32
    x = x_ref[0]
    rbd = rbd_ref[...]
    rbdt = rbdt_ref[...]
    sbd = sbd_ref[...]

    v = jnp.sum(x * x, axis=0, keepdims=True)
    v_t = v.reshape(_DIM, 1)
    ii = jax.lax.broadcasted_iota(jnp.int32, (_DIM, _DIM), 0)
    jj = jax.lax.broadcasted_iota(jnp.int32, (_DIM, _DIM), 1)
    gt = (v_t > v).astype(f32)
    eq = ((v_t == v) & (ii < jj)).astype(f32)
    rank = jnp.sum(gt + eq, axis=0, keepdims=True)
    is_out = rank < float(_NOUT)
    mask_out = is_out.astype(f32)
    mask_reg = 1.0 - mask_out

    lower = (ii < jj).astype(f32)
    pos_reg = jnp.sum(mask_reg.reshape(_DIM, 1) * lower, axis=0, keepdims=True)
    pos_out = jnp.sum(mask_out.reshape(_DIM, 1) * lower, axis=0, keepdims=True)
    target = jnp.where(is_out, pos_out + float(_NREG), pos_reg)

    onehot = (target.reshape(_DIM, 1) == jj.astype(f32)).astype(f32)

    g = _dot(x, onehot, _EXACT)

    jlane = jax.lax.broadcasted_iota(jnp.int32, (1, _DIM), 1)
    colreg = (jlane < _NREG).astype(f32)
    colout = 1.0 - colreg

    gsq = g * g
    ss_r = jnp.sum(gsq[:, :_NREG], axis=1, keepdims=True)
    ss_o = jnp.sum(gsq[:, _NREG:], axis=1, keepdims=True)
    norm_r = jnp.maximum(jnp.sqrt(ss_r), 1e-8)
    norm_o = jnp.maximum(jnp.sqrt(ss_o), 1e-8)
    norms = colreg * norm_r + colout * norm_o

    xn = g / norms
    y = _dot(xn, rbdt, _MM)
    yh1 = jnp.where(y <= 0.0, -_L1, _L1)
    yh2 = jnp.where(y <= -_MID2, -_L2B,
                    jnp.where(y <= 0.0, -_L2A,
                              jnp.where(y <= _MID2, _L2A, _L2B)))
    y_hat = colreg * yh1 + colout * yh2
    x_mse = _dot(y_hat, rbd, _MM) * norms

    resid = g - x_mse
    rsq = resid * resid
    rs_r = jnp.sum(rsq[:, :_NREG], axis=1, keepdims=True)
    rs_o = jnp.sum(rsq[:, _NREG:], axis=1, keepdims=True)
    rn_r = jnp.maximum(jnp.sqrt(rs_r), 1e-10)
    rn_o = jnp.maximum(jnp.sqrt(rs_o), 1e-10)
    rnorms = colreg * rn_r + colout * rn_o
    rn = resid / rnorms
    proj = _dot_t(rn, sbd, _MM)
    signs = jnp.where(proj >= 0.0, 1.0, -1.0)
    kscale = colreg * _K_REG + colout * _K_OUT
    r_hat = (kscale * _dot(signs, sbd, _MM)) * rnorms

    values = x_mse + r_hat
    o_ref[0] = _dot_t(values, onehot, _EXACT)


@jax.jit
def kernel(x, rotation_reg, S_reg, rotation_out, S_out):
    b, h, n, d = x.shape
    xr = x.reshape(b * h, n, d)
    rbd = jnp.zeros((d, d), jnp.float32)
    rbd = rbd.at[:_NREG, :_NREG].set(rotation_reg).at[_NREG:, _NREG:].set(rotation_out)
    sbd = jnp.zeros((d, d), jnp.float32)
    sbd = sbd.at[:_NREG, :_NREG].set(S_reg).at[_NREG:, _NREG:].set(S_out)
    rbdt = rbd.T
    out = pl.pallas_call(
        _tq_kernel,
        grid=(b * h,),
        in_specs=[
            pl.BlockSpec((1, n, d), lambda i: (i, 0, 0)),
            pl.BlockSpec((d, d), lambda i: (0, 0)),
            pl.BlockSpec((d, d), lambda i: (0, 0)),
            pl.BlockSpec((d, d), lambda i: (0, 0)),
        ],
        out_specs=pl.BlockSpec((1, n, d), lambda i: (i, 0, 0)),
        out_shape=jax.ShapeDtypeStruct((b * h, n, d), jnp.float32),
    )(xr, rbd, rbdt, sbd)
    return out.reshape(b, h, n, d)
